# Initial kernel scaffold; baseline (speedup 1.0000x reference)
import jax
import jax.numpy as jnp
from jax.experimental import pallas as pl


def kernel(X, muK, cK):
    raise NotImplementedError("write your pallas kernel here")


# speedup vs baseline: 121.3529x; 121.3529x over previous
import functools

import jax
import jax.numpy as jnp
from jax.experimental import pallas as pl
from jax.experimental.pallas import tpu as pltpu

TQ = 512


def _ncm_kernel(x_ref, mut_ref, ck_ref, out_ref, mu_norm_ref):
    @pl.when(pl.program_id(0) == 0)
    def _():
        m32 = mut_ref[...].astype(jnp.float32)
        mu_norm_ref[...] = jnp.sum(m32 * m32, axis=0, keepdims=True)

    x = x_ref[...]
    p = jax.lax.dot_general(
        x.astype(jnp.bfloat16), mut_ref[...],
        dimension_numbers=(((1,), (0,)), ((), ())),
        preferred_element_type=jnp.float32,
    )
    xn = jnp.sum(x * x, axis=1, keepdims=True)
    scores = (p + p) - xn - mu_norm_ref[...]
    min_col = jnp.min(scores, axis=1, keepdims=True) - 1.0
    not_visited = ck_ref[...] == 0.0
    out_ref[...] = jnp.where(not_visited, min_col, scores)


@jax.jit
def kernel(X, muK, cK):
    Q, D = X.shape
    K = muK.shape[0]
    muT = muK.T.astype(jnp.bfloat16)
    cK2 = cK.reshape(1, K)
    grid = (Q // TQ,)
    return pl.pallas_call(
        _ncm_kernel,
        grid=grid,
        in_specs=[
            pl.BlockSpec((TQ, D), lambda i: (i, 0)),
            pl.BlockSpec((D, K), lambda i: (0, 0)),
            pl.BlockSpec((1, K), lambda i: (0, 0)),
        ],
        out_specs=pl.BlockSpec((TQ, K), lambda i: (i, 0)),
        out_shape=jax.ShapeDtypeStruct((Q, K), jnp.float32),
        scratch_shapes=[pltpu.VMEM((1, K), jnp.float32)],
        compiler_params=pltpu.CompilerParams(
            dimension_semantics=("arbitrary",),
        ),
    )(X, muT, cK2)


# speedup vs baseline: 134.1466x; 1.1054x over previous
import jax
import jax.numpy as jnp
from jax.experimental import pallas as pl
from jax.experimental.pallas import tpu as pltpu

TQ = 512


def _ncm_kernel(x_ref, mu_ref, ck_ref, out_ref, mu2_ref, mu_norm_ref):
    @pl.when(pl.program_id(0) == 0)
    def _():
        m = mu_ref[...]
        mu2_ref[...] = (m + m).astype(jnp.bfloat16)
        mu_norm_ref[...] = jax.lax.dot_general(
            jnp.ones((1, m.shape[1]), jnp.bfloat16), (m * m).astype(jnp.bfloat16),
            dimension_numbers=(((1,), (1,)), ((), ())),
            preferred_element_type=jnp.float32,
        )

    x = x_ref[...]
    p = jax.lax.dot_general(
        x.astype(jnp.bfloat16), mu2_ref[...],
        dimension_numbers=(((1,), (1,)), ((), ())),
        preferred_element_type=jnp.float32,
    )
    xn = jnp.sum(x * x, axis=1, keepdims=True)
    scores = p - xn - mu_norm_ref[...]
    min_col = jnp.min(scores, axis=1, keepdims=True) - 1.0
    not_visited = ck_ref[...] == 0.0
    out_ref[...] = jnp.where(not_visited, min_col, scores)


@jax.jit
def kernel(X, muK, cK):
    Q, D = X.shape
    K = muK.shape[0]
    cK2 = cK.reshape(1, K)
    grid = (Q // TQ,)
    return pl.pallas_call(
        _ncm_kernel,
        grid=grid,
        in_specs=[
            pl.BlockSpec((TQ, D), lambda i: (i, 0)),
            pl.BlockSpec((K, D), lambda i: (0, 0)),
            pl.BlockSpec((1, K), lambda i: (0, 0)),
        ],
        out_specs=pl.BlockSpec((TQ, K), lambda i: (i, 0)),
        out_shape=jax.ShapeDtypeStruct((Q, K), jnp.float32),
        scratch_shapes=[
            pltpu.VMEM((K, D), jnp.bfloat16),
            pltpu.VMEM((1, K), jnp.float32),
        ],
        compiler_params=pltpu.CompilerParams(
            dimension_semantics=("arbitrary",),
        ),
    )(X, muK, cK2)


# speedup vs baseline: 204.9202x; 1.5276x over previous
import jax
import jax.numpy as jnp
from jax.experimental import pallas as pl
from jax.experimental.pallas import tpu as pltpu

TQ = 512


def _ncm_kernel(x_ref, mu_ref, ck_ref, out_ref, mu2_ref, mu_norm_ref):
    @pl.when(pl.program_id(0) == 0)
    def _():
        m = mu_ref[...]
        mu2_ref[...] = (m + m).astype(jnp.bfloat16)
        mu_norm_ref[...] = jnp.sum(m * m, axis=1, keepdims=True)

    x = x_ref[...]
    p = jax.lax.dot_general(
        mu2_ref[...], x.astype(jnp.bfloat16),
        dimension_numbers=(((1,), (1,)), ((), ())),
        preferred_element_type=jnp.float32,
    )
    ones_row = jnp.ones((1, x.shape[1]), jnp.bfloat16)
    xn = jax.lax.dot_general(
        ones_row, (x * x).astype(jnp.bfloat16),
        dimension_numbers=(((1,), (1,)), ((), ())),
        preferred_element_type=jnp.float32,
    )
    t = p - mu_norm_ref[...]
    min_row = jnp.min(t, axis=0, keepdims=True) - 1.0
    not_visited = ck_ref[...] == 0.0
    out_ref[...] = jnp.where(not_visited, min_row, t) - xn


@jax.jit
def kernel(X, muK, cK):
    Q, D = X.shape
    K = muK.shape[0]
    cK2 = cK.reshape(K, 1)
    grid = (Q // TQ,)
    out_t = pl.pallas_call(
        _ncm_kernel,
        grid=grid,
        in_specs=[
            pl.BlockSpec((TQ, D), lambda i: (i, 0)),
            pl.BlockSpec((K, D), lambda i: (0, 0)),
            pl.BlockSpec((K, 1), lambda i: (0, 0)),
        ],
        out_specs=pl.BlockSpec((K, TQ), lambda i: (0, i)),
        out_shape=jax.ShapeDtypeStruct((K, Q), jnp.float32),
        scratch_shapes=[
            pltpu.VMEM((K, D), jnp.bfloat16),
            pltpu.VMEM((K, 1), jnp.float32),
        ],
        compiler_params=pltpu.CompilerParams(
            dimension_semantics=("arbitrary",),
        ),
    )(X, muK, cK2)
    return out_t.T


# speedup vs baseline: 216.7681x; 1.0578x over previous
import jax
import jax.numpy as jnp
from jax.experimental import pallas as pl
from jax.experimental.pallas import tpu as pltpu

TQ = 1024


def _ncm_kernel(x_ref, mu_ref, ck_ref, out_ref, mu2_ref, mu_norm_ref):
    @pl.when(pl.program_id(0) == 0)
    def _():
        m = mu_ref[...]
        mu2_ref[...] = (m + m).astype(jnp.bfloat16)
        mu_norm_ref[...] = jnp.sum(m * m, axis=1, keepdims=True)

    xb = x_ref[...].astype(jnp.bfloat16)
    p = jax.lax.dot_general(
        mu2_ref[...], xb,
        dimension_numbers=(((1,), (1,)), ((), ())),
        preferred_element_type=jnp.float32,
    )
    ones_row = jnp.ones((1, xb.shape[1]), jnp.bfloat16)
    xn = jax.lax.dot_general(
        ones_row, xb * xb,
        dimension_numbers=(((1,), (1,)), ((), ())),
        preferred_element_type=jnp.float32,
    )
    t = p - mu_norm_ref[...]
    min_row = jnp.min(t, axis=0, keepdims=True) - 1.0
    not_visited = ck_ref[...] == 0.0
    out_ref[...] = jnp.where(not_visited, min_row, t) - xn


@jax.jit
def kernel(X, muK, cK):
    Q, D = X.shape
    K = muK.shape[0]
    cK2 = cK.reshape(K, 1)
    grid = (Q // TQ,)
    out_t = pl.pallas_call(
        _ncm_kernel,
        grid=grid,
        in_specs=[
            pl.BlockSpec((TQ, D), lambda i: (i, 0)),
            pl.BlockSpec((K, D), lambda i: (0, 0)),
            pl.BlockSpec((K, 1), lambda i: (0, 0)),
        ],
        out_specs=pl.BlockSpec((K, TQ), lambda i: (0, i)),
        out_shape=jax.ShapeDtypeStruct((K, Q), jnp.float32),
        scratch_shapes=[
            pltpu.VMEM((K, D), jnp.bfloat16),
            pltpu.VMEM((K, 1), jnp.float32),
        ],
        compiler_params=pltpu.CompilerParams(
            dimension_semantics=("arbitrary",),
        ),
    )(X, muK, cK2)
    return out_t.T
